# baseline (speedup 1.0000x reference)
import functools

import jax
import jax.numpy as jnp
from jax import lax
from jax.experimental import pallas as pl
from jax.experimental.pallas import tpu as pltpu
from jax.experimental.pallas import tpu_sc as plsc

B, N, IN_DIM, OUT_DIM, K = 8, 4096, 3, 64, 16
MID = OUT_DIM // 2
S = N // 2
EPS = 1e-5
BN_TOT = B * N
BS_TOT = B * S
TW = 128
FOFF = 16
SQ = 256

NW = 32
QPW = BS_TOT // NW
IPW = QPW * K
QPC = 8
IPC = QPC * K
NCHUNK = QPW // QPC


def _mlp1_body(x_ref, w1_ref, b1_ref, g1_ref, bt1_ref, w2_ref, b2_ref,
               out_ref):
    xc = x_ref[...]
    h = lax.dot_general(w1_ref[...], xc, (((1,), (0,)), ((), ())),
                        preferred_element_type=jnp.float32)
    h = h + b1_ref[...]
    m = jnp.mean(h, axis=1, keepdims=True)
    v = jnp.mean((h - m) ** 2, axis=1, keepdims=True)
    h = (h - m) / jnp.sqrt(v + EPS) * g1_ref[...] + bt1_ref[...]
    h = jnp.maximum(h, 0.0)
    f = lax.dot_general(w2_ref[...], h, (((1,), (0,)), ((), ())),
                        preferred_element_type=jnp.float32)
    f = f + b2_ref[...]
    z = jnp.zeros((BN_TOT, FOFF - 3), jnp.float32)
    z2 = jnp.zeros((BN_TOT, TW - FOFF - OUT_DIM), jnp.float32)
    out_ref[...] = jnp.concatenate([xc.T, z, f.T, z2], axis=1)


def _mlp1(xc, W1, b1, g1, bt1, W2, b2):
    return pl.pallas_call(
        _mlp1_body,
        out_shape=jax.ShapeDtypeStruct((BN_TOT, TW), jnp.float32),
    )(xc, W1, b1.reshape(MID, 1), g1.reshape(MID, 1), bt1.reshape(MID, 1),
      W2, b2.reshape(OUT_DIM, 1))


def _fps_body(xt_ref, nx_ref):
    x0 = xt_ref[0]
    x1 = xt_ref[1]
    x2 = xt_ref[2]
    lanesf = lax.broadcasted_iota(jnp.int32, (B, N), 1).astype(jnp.float32)
    G = N // 128

    def pick(cond, a, b):
        return tuple(jnp.where(cond, xa, xb) for xa, xb in zip(a, b))

    def combine(a, b):
        takeb = (b[0] > a[0]) | ((b[0] == a[0]) & (b[1] < a[1]))
        return pick(takeb, b, a)

    def body(i, carry):
        dist, c0, c1, c2 = carry
        nx_ref[pl.ds(i, 1), :, :] = jnp.concatenate(
            [c0, c1, c2], axis=1)[None]
        d = ((x0 - c0) ** 2 + (x1 - c1) ** 2) + (x2 - c2) ** 2
        dist = jnp.minimum(dist, d)
        t = (dist, lanesf, x0, x1, x2)
        w = N
        while w > 128:
            w //= 2
            t = combine(tuple(v[:, :w] for v in t),
                        tuple(v[:, w:] for v in t))
        sh = 64
        while sh >= 1:
            t = combine(t, tuple(pltpu.roll(v, sh, 1) for v in t))
            sh //= 2
        return dist, t[2][:, 0:1], t[3][:, 0:1], t[4][:, 0:1]

    dist0 = jnp.full((B, N), 1e10, jnp.float32)
    lax.fori_loop(0, S, body,
                  (dist0, x0[:, 0:1], x1[:, 0:1], x2[:, 0:1]))


def _fps(xt):
    return pl.pallas_call(
        _fps_body,
        out_shape=jax.ShapeDtypeStruct((S, B, 3), jnp.float32),
    )(xt)


def _knn_body(nx_ref, x_ref, xt_ref, idx_ref):
    b = pl.program_id(0)
    q = nx_ref[0]
    xb = x_ref[0]
    xt = xt_ref[0]
    ab = lax.dot_general(q, xb, (((1,), (1,)), ((), ())),
                         preferred_element_type=jnp.float32)
    qsq = jnp.sum(q * q, axis=1, keepdims=True)
    xsq = jnp.sum(xt * xt, axis=0, keepdims=True)
    sq = (-2.0 * ab + qsq) + xsq
    lanesf = lax.broadcasted_iota(jnp.int32, (SQ, N), 1).astype(jnp.float32)
    boff = b * N
    for k in range(K):
        mn = jnp.min(sq, axis=1, keepdims=True)
        jf = jnp.min(jnp.where(sq == mn, lanesf, float(N)),
                     axis=1, keepdims=True)
        idx_ref[0, :, pl.ds(k, 1)] = jf.astype(jnp.int32) + boff
        sq = jnp.where(lanesf == jf, jnp.inf, sq)


def _knn(new_xyz, x_in, xtb):
    return pl.pallas_call(
        _knn_body,
        grid=(B, S // SQ),
        in_specs=[
            pl.BlockSpec((1, SQ, 3), lambda b, t: (b, t, 0)),
            pl.BlockSpec((1, N, 3), lambda b, t: (b, 0, 0)),
            pl.BlockSpec((1, 3, N), lambda b, t: (b, 0, 0)),
        ],
        out_specs=pl.BlockSpec((1, SQ, K), lambda b, t: (b, t, 0)),
        out_shape=jax.ShapeDtypeStruct((B, S, K), jnp.int32),
    )(new_xyz, x_in, xtb)


def _scpool_body(tab_hbm, idx_hbm, out_hbm, idx_v, gt, ot, sem):
    wid = lax.axis_index("s") * 2 + lax.axis_index("c")
    base_q = wid * QPW
    pltpu.sync_copy(idx_hbm.at[pl.ds(wid * NCHUNK, NCHUNK)], idx_v)

    def chunk(ck, carry):
        irow = idx_v.at[ck]
        pltpu.async_copy(tab_hbm.at[irow], gt, sem).wait()
        for q in range(QPC):
            orow = ck * QPC + q
            for cc in range((FOFF + OUT_DIM) // 16):
                m = gt[q * K, pl.ds(cc * 16, 16)]
                for r in range(1, K):
                    m = jnp.maximum(m, gt[q * K + r, pl.ds(cc * 16, 16)])
                ot[orow, pl.ds(cc * 16, 16)] = m
        return carry

    lax.fori_loop(0, NCHUNK, chunk, 0)
    pltpu.sync_copy(ot, out_hbm.at[pl.ds(base_q, QPW)])


@functools.cache
def _scpool_build():
    return pl.kernel(
        _scpool_body,
        out_type=jax.ShapeDtypeStruct((BS_TOT, TW), jnp.float32),
        mesh=plsc.VectorSubcoreMesh(core_axis_name="c", subcore_axis_name="s"),
        scratch_types=[
            pltpu.VMEM((NCHUNK, IPC), jnp.int32),
            pltpu.VMEM((IPC, TW), jnp.float32),
            pltpu.VMEM((QPW, TW), jnp.float32),
            pltpu.SemaphoreType.DMA,
        ],
    )


def _scpool(tab, idxf):
    return _scpool_build()(tab, idxf)


def _mlp2_body(pool_ref, nx_ref, w3_ref, b3_ref, g2_ref, bt2_ref,
               w4_ref, b4_ref, out_ref):
    xyz = pool_ref[:, 0:3] - nx_ref[...]
    xm = jnp.concatenate([xyz, pool_ref[:, FOFF:FOFF + OUT_DIM]], axis=1)
    h = lax.dot_general(xm, w3_ref[...], (((1,), (1,)), ((), ())),
                        preferred_element_type=jnp.float32)
    h = h + b3_ref[...]
    m = jnp.mean(h, axis=0, keepdims=True)
    v = jnp.mean((h - m) ** 2, axis=0, keepdims=True)
    h = (h - m) / jnp.sqrt(v + EPS) * g2_ref[...] + bt2_ref[...]
    h = jnp.maximum(h, 0.0)
    o = lax.dot_general(h, w4_ref[...], (((1,), (1,)), ((), ())),
                        preferred_element_type=jnp.float32)
    out_ref[...] = o + b4_ref[...]


def _mlp2(pool, nx, W3, b3, g2, bt2, W4, b4):
    return pl.pallas_call(
        _mlp2_body,
        out_shape=jax.ShapeDtypeStruct((BS_TOT, OUT_DIM), jnp.float32),
    )(pool, nx, W3, b3.reshape(1, OUT_DIM), g2.reshape(1, OUT_DIM),
      bt2.reshape(1, OUT_DIM), W4, b4.reshape(1, OUT_DIM))


def kernel(x_in, W1, b1, g1, bt1, W2, b2, W3, b3, g2, bt2, W4, b4):
    xt = jnp.transpose(x_in, (2, 0, 1))
    tab = _mlp1(xt.reshape(3, BN_TOT), W1, b1, g1, bt1, W2, b2)
    nxt = _fps(xt)
    new_xyz = jnp.transpose(nxt, (1, 0, 2))
    idx = _knn(new_xyz, x_in, jnp.transpose(x_in, (0, 2, 1)))
    pool = _scpool(tab, idx.reshape(NW * NCHUNK, IPC))
    out = _mlp2(pool, new_xyz.reshape(BS_TOT, 3),
                W3, b3, g2, bt2, W4, b4)
    return out.reshape(B, S, OUT_DIM)


# speedup vs baseline: 1.2544x; 1.2544x over previous
import functools

import jax
import jax.numpy as jnp
from jax import lax
from jax.experimental import pallas as pl
from jax.experimental.pallas import tpu as pltpu
from jax.experimental.pallas import tpu_sc as plsc

B, N, IN_DIM, OUT_DIM, K = 8, 4096, 3, 64, 16
MID = OUT_DIM // 2
S = N // 2
EPS = 1e-5
BN_TOT = B * N
BS_TOT = B * S
TW = 128
FOFF = 16
SQ = 256

NW = 32
QPW = BS_TOT // NW
IPW = QPW * K
QPC = 8
IPC = QPC * K
NCHUNK = QPW // QPC


def _mlp1_body(x_ref, w1_ref, b1_ref, g1_ref, bt1_ref, w2_ref, b2_ref,
               out_ref):
    xc = x_ref[...]
    h = lax.dot_general(w1_ref[...], xc, (((1,), (0,)), ((), ())),
                        preferred_element_type=jnp.float32)
    h = h + b1_ref[...]
    m = jnp.mean(h, axis=1, keepdims=True)
    v = jnp.mean((h - m) ** 2, axis=1, keepdims=True)
    h = (h - m) / jnp.sqrt(v + EPS) * g1_ref[...] + bt1_ref[...]
    h = jnp.maximum(h, 0.0)
    f = lax.dot_general(w2_ref[...], h, (((1,), (0,)), ((), ())),
                        preferred_element_type=jnp.float32)
    f = f + b2_ref[...]
    z = jnp.zeros((BN_TOT, FOFF - 3), jnp.float32)
    z2 = jnp.zeros((BN_TOT, TW - FOFF - OUT_DIM), jnp.float32)
    out_ref[...] = jnp.concatenate([xc.T, z, f.T, z2], axis=1)


def _mlp1(xc, W1, b1, g1, bt1, W2, b2):
    return pl.pallas_call(
        _mlp1_body,
        out_shape=jax.ShapeDtypeStruct((BN_TOT, TW), jnp.float32),
    )(xc, W1, b1.reshape(MID, 1), g1.reshape(MID, 1), bt1.reshape(MID, 1),
      W2, b2.reshape(OUT_DIM, 1))


def _fps_body(xt_ref, nx_ref):
    x0 = xt_ref[0]
    x1 = xt_ref[1]
    x2 = xt_ref[2]
    lanesf = lax.broadcasted_iota(jnp.int32, (B, N), 1).astype(jnp.float32)

    def body(i, carry):
        dist, c0, c1, c2 = carry
        nx_ref[pl.ds(i, 1), :, :] = jnp.concatenate(
            [c0, c1, c2], axis=1)[None]
        d = ((x0 - c0) ** 2 + (x1 - c1) ** 2) + (x2 - c2) ** 2
        dist = jnp.minimum(dist, d)
        mx = jnp.max(dist, axis=1, keepdims=True)
        jf = jnp.min(jnp.where(dist == mx, lanesf, float(N)),
                     axis=1, keepdims=True)
        sel = lanesf == jf
        c0 = jnp.sum(jnp.where(sel, x0, 0.0), axis=1, keepdims=True)
        c1 = jnp.sum(jnp.where(sel, x1, 0.0), axis=1, keepdims=True)
        c2 = jnp.sum(jnp.where(sel, x2, 0.0), axis=1, keepdims=True)
        return dist, c0, c1, c2

    dist0 = jnp.full((B, N), 1e10, jnp.float32)
    lax.fori_loop(0, S, body,
                  (dist0, x0[:, 0:1], x1[:, 0:1], x2[:, 0:1]))


def _fps(xt):
    return pl.pallas_call(
        _fps_body,
        out_shape=jax.ShapeDtypeStruct((S, B, 3), jnp.float32),
    )(xt)


def _knn_body(nx_ref, x_ref, xt_ref, idx_ref):
    b = pl.program_id(0)
    q = nx_ref[0]
    xb = x_ref[0]
    xt = xt_ref[0]
    ab = lax.dot_general(q, xb, (((1,), (1,)), ((), ())),
                         preferred_element_type=jnp.float32)
    qsq = jnp.sum(q * q, axis=1, keepdims=True)
    xsq = jnp.sum(xt * xt, axis=0, keepdims=True)
    sq = (-2.0 * ab + qsq) + xsq
    lanesf = lax.broadcasted_iota(jnp.int32, (SQ, N), 1).astype(jnp.float32)
    boff = b * N
    for k in range(K):
        mn = jnp.min(sq, axis=1, keepdims=True)
        jf = jnp.min(jnp.where(sq == mn, lanesf, float(N)),
                     axis=1, keepdims=True)
        idx_ref[0, :, pl.ds(k, 1)] = jf.astype(jnp.int32) + boff
        sq = jnp.where(lanesf == jf, jnp.inf, sq)


def _knn(new_xyz, x_in, xtb):
    return pl.pallas_call(
        _knn_body,
        grid=(B, S // SQ),
        in_specs=[
            pl.BlockSpec((1, SQ, 3), lambda b, t: (b, t, 0)),
            pl.BlockSpec((1, N, 3), lambda b, t: (b, 0, 0)),
            pl.BlockSpec((1, 3, N), lambda b, t: (b, 0, 0)),
        ],
        out_specs=pl.BlockSpec((1, SQ, K), lambda b, t: (b, t, 0)),
        out_shape=jax.ShapeDtypeStruct((B, S, K), jnp.int32),
    )(new_xyz, x_in, xtb)


def _scpool_body(tab_hbm, idx_hbm, out_hbm, idx_v, gt, ot, sem):
    wid = lax.axis_index("s") * 2 + lax.axis_index("c")
    base_q = wid * QPW
    pltpu.sync_copy(idx_hbm.at[pl.ds(wid * NCHUNK, NCHUNK)], idx_v)

    def chunk(ck, carry):
        irow = idx_v.at[ck]
        pltpu.async_copy(tab_hbm.at[irow], gt, sem).wait()
        for q in range(QPC):
            orow = ck * QPC + q
            for cc in range((FOFF + OUT_DIM) // 16):
                m = gt[q * K, pl.ds(cc * 16, 16)]
                for r in range(1, K):
                    m = jnp.maximum(m, gt[q * K + r, pl.ds(cc * 16, 16)])
                ot[orow, pl.ds(cc * 16, 16)] = m
        return carry

    lax.fori_loop(0, NCHUNK, chunk, 0)
    pltpu.sync_copy(ot, out_hbm.at[pl.ds(base_q, QPW)])


@functools.cache
def _scpool_build():
    return pl.kernel(
        _scpool_body,
        out_type=jax.ShapeDtypeStruct((BS_TOT, TW), jnp.float32),
        mesh=plsc.VectorSubcoreMesh(core_axis_name="c", subcore_axis_name="s"),
        scratch_types=[
            pltpu.VMEM((NCHUNK, IPC), jnp.int32),
            pltpu.VMEM((IPC, TW), jnp.float32),
            pltpu.VMEM((QPW, TW), jnp.float32),
            pltpu.SemaphoreType.DMA,
        ],
    )


def _scpool(tab, idxf):
    return _scpool_build()(tab, idxf)


def _mlp2_body(pool_ref, nx_ref, w3_ref, b3_ref, g2_ref, bt2_ref,
               w4_ref, b4_ref, out_ref):
    xyz = pool_ref[:, 0:3] - nx_ref[...]
    xm = jnp.concatenate([xyz, pool_ref[:, FOFF:FOFF + OUT_DIM]], axis=1)
    h = lax.dot_general(xm, w3_ref[...], (((1,), (1,)), ((), ())),
                        preferred_element_type=jnp.float32)
    h = h + b3_ref[...]
    m = jnp.mean(h, axis=0, keepdims=True)
    v = jnp.mean((h - m) ** 2, axis=0, keepdims=True)
    h = (h - m) / jnp.sqrt(v + EPS) * g2_ref[...] + bt2_ref[...]
    h = jnp.maximum(h, 0.0)
    o = lax.dot_general(h, w4_ref[...], (((1,), (1,)), ((), ())),
                        preferred_element_type=jnp.float32)
    out_ref[...] = o + b4_ref[...]


def _mlp2(pool, nx, W3, b3, g2, bt2, W4, b4):
    return pl.pallas_call(
        _mlp2_body,
        out_shape=jax.ShapeDtypeStruct((BS_TOT, OUT_DIM), jnp.float32),
    )(pool, nx, W3, b3.reshape(1, OUT_DIM), g2.reshape(1, OUT_DIM),
      bt2.reshape(1, OUT_DIM), W4, b4.reshape(1, OUT_DIM))


def kernel(x_in, W1, b1, g1, bt1, W2, b2, W3, b3, g2, bt2, W4, b4):
    xt = jnp.transpose(x_in, (2, 0, 1))
    tab = _mlp1(xt.reshape(3, BN_TOT), W1, b1, g1, bt1, W2, b2)
    nxt = _fps(xt)
    new_xyz = jnp.transpose(nxt, (1, 0, 2))
    idx = _knn(new_xyz, x_in, jnp.transpose(x_in, (0, 2, 1)))
    pool = _scpool(tab, idx.reshape(NW * NCHUNK, IPC))
    out = _mlp2(pool, new_xyz.reshape(BS_TOT, 3),
                W3, b3, g2, bt2, W4, b4)
    return out.reshape(B, S, OUT_DIM)


# speedup vs baseline: 1.2545x; 1.0000x over previous
import functools

import jax
import jax.numpy as jnp
from jax import lax
from jax.experimental import pallas as pl
from jax.experimental.pallas import tpu as pltpu
from jax.experimental.pallas import tpu_sc as plsc

B, N, IN_DIM, OUT_DIM, K = 8, 4096, 3, 64, 16
MID = OUT_DIM // 2
S = N // 2
EPS = 1e-5
BN_TOT = B * N
BS_TOT = B * S
TW = 128
FOFF = 16
SQ = 256

NW = 32
QPW = BS_TOT // NW
IPW = QPW * K
QPC = 8
IPC = QPC * K
NCHUNK = QPW // QPC


def _mlp1_body(x_ref, w1_ref, b1_ref, g1_ref, bt1_ref, w2_ref, b2_ref,
               out_ref):
    xc = x_ref[...]
    h = lax.dot_general(w1_ref[...], xc, (((1,), (0,)), ((), ())),
                        preferred_element_type=jnp.float32)
    h = h + b1_ref[...]
    m = jnp.mean(h, axis=1, keepdims=True)
    v = jnp.mean((h - m) ** 2, axis=1, keepdims=True)
    h = (h - m) / jnp.sqrt(v + EPS) * g1_ref[...] + bt1_ref[...]
    h = jnp.maximum(h, 0.0)
    f = lax.dot_general(w2_ref[...], h, (((1,), (0,)), ((), ())),
                        preferred_element_type=jnp.float32)
    f = f + b2_ref[...]
    z = jnp.zeros((BN_TOT, FOFF - 3), jnp.float32)
    z2 = jnp.zeros((BN_TOT, TW - FOFF - OUT_DIM), jnp.float32)
    out_ref[...] = jnp.concatenate([xc.T, z, f.T, z2], axis=1)


def _mlp1(xc, W1, b1, g1, bt1, W2, b2):
    return pl.pallas_call(
        _mlp1_body,
        out_shape=jax.ShapeDtypeStruct((BN_TOT, TW), jnp.float32),
    )(xc, W1, b1.reshape(MID, 1), g1.reshape(MID, 1), bt1.reshape(MID, 1),
      W2, b2.reshape(OUT_DIM, 1))


def _fps_body(xt_ref, nx_ref):
    x0 = xt_ref[0]
    x1 = xt_ref[1]
    x2 = xt_ref[2]
    lanesf = lax.broadcasted_iota(jnp.int32, (B, N), 1).astype(jnp.float32)

    def body(i, carry):
        dist, c0, c1, c2 = carry
        nx_ref[pl.ds(i, 1), :, :] = jnp.concatenate(
            [c0, c1, c2], axis=1)[None]
        d = ((x0 - c0) ** 2 + (x1 - c1) ** 2) + (x2 - c2) ** 2
        dist = jnp.minimum(dist, d)
        mx = jnp.max(dist, axis=1, keepdims=True)
        jf = jnp.min(jnp.where(dist == mx, lanesf, float(N)),
                     axis=1, keepdims=True)
        sel = lanesf == jf
        c0 = jnp.sum(jnp.where(sel, x0, 0.0), axis=1, keepdims=True)
        c1 = jnp.sum(jnp.where(sel, x1, 0.0), axis=1, keepdims=True)
        c2 = jnp.sum(jnp.where(sel, x2, 0.0), axis=1, keepdims=True)
        return dist, c0, c1, c2

    dist0 = jnp.full((B, N), 1e10, jnp.float32)
    lax.fori_loop(0, S, body,
                  (dist0, x0[:, 0:1], x1[:, 0:1], x2[:, 0:1]))


def _fps(xt):
    return pl.pallas_call(
        _fps_body,
        out_shape=jax.ShapeDtypeStruct((S, B, 3), jnp.float32),
    )(xt)


def _knn_body(nx_ref, x_ref, xt_ref, idx_ref):
    b = pl.program_id(0)
    q = nx_ref[0]
    xb = x_ref[0]
    xt = xt_ref[0]
    ab = lax.dot_general(q, xb, (((1,), (1,)), ((), ())),
                         preferred_element_type=jnp.float32)
    qsq = jnp.sum(q * q, axis=1, keepdims=True)
    xsq = jnp.sum(xt * xt, axis=0, keepdims=True)
    sq = (-2.0 * ab + qsq) + xsq
    lanesf = lax.broadcasted_iota(jnp.int32, (SQ, N), 1).astype(jnp.float32)
    boff = b * N
    for k in range(K):
        mn = jnp.min(sq, axis=1, keepdims=True)
        jf = jnp.min(jnp.where(sq == mn, lanesf, float(N)),
                     axis=1, keepdims=True)
        idx_ref[0, :, pl.ds(k, 1)] = jf.astype(jnp.int32) + boff
        sq = jnp.where(lanesf == jf, jnp.inf, sq)


def _knn(new_xyz, x_in, xtb):
    return pl.pallas_call(
        _knn_body,
        grid=(B, S // SQ),
        in_specs=[
            pl.BlockSpec((1, SQ, 3), lambda b, t: (b, t, 0)),
            pl.BlockSpec((1, N, 3), lambda b, t: (b, 0, 0)),
            pl.BlockSpec((1, 3, N), lambda b, t: (b, 0, 0)),
        ],
        out_specs=pl.BlockSpec((1, SQ, K), lambda b, t: (b, t, 0)),
        out_shape=jax.ShapeDtypeStruct((B, S, K), jnp.int32),
        compiler_params=pltpu.CompilerParams(
            dimension_semantics=("parallel", "parallel")),
    )(new_xyz, x_in, xtb)


def _scpool_body(tab_hbm, idx_hbm, out_hbm, idx_v, gt, ot, sem):
    wid = lax.axis_index("s") * 2 + lax.axis_index("c")
    base_q = wid * QPW
    pltpu.sync_copy(idx_hbm.at[pl.ds(wid * NCHUNK, NCHUNK)], idx_v)

    def chunk(ck, carry):
        irow = idx_v.at[ck]
        pltpu.async_copy(tab_hbm.at[irow], gt, sem).wait()
        for q in range(QPC):
            orow = ck * QPC + q
            for cc in range((FOFF + OUT_DIM) // 16):
                m = gt[q * K, pl.ds(cc * 16, 16)]
                for r in range(1, K):
                    m = jnp.maximum(m, gt[q * K + r, pl.ds(cc * 16, 16)])
                ot[orow, pl.ds(cc * 16, 16)] = m
        return carry

    lax.fori_loop(0, NCHUNK, chunk, 0)
    pltpu.sync_copy(ot, out_hbm.at[pl.ds(base_q, QPW)])


@functools.cache
def _scpool_build():
    return pl.kernel(
        _scpool_body,
        out_type=jax.ShapeDtypeStruct((BS_TOT, TW), jnp.float32),
        mesh=plsc.VectorSubcoreMesh(core_axis_name="c", subcore_axis_name="s"),
        scratch_types=[
            pltpu.VMEM((NCHUNK, IPC), jnp.int32),
            pltpu.VMEM((IPC, TW), jnp.float32),
            pltpu.VMEM((QPW, TW), jnp.float32),
            pltpu.SemaphoreType.DMA,
        ],
    )


def _scpool(tab, idxf):
    return _scpool_build()(tab, idxf)


def _mlp2_body(pool_ref, nx_ref, w3_ref, b3_ref, g2_ref, bt2_ref,
               w4_ref, b4_ref, out_ref):
    xyz = pool_ref[:, 0:3] - nx_ref[...]
    xm = jnp.concatenate([xyz, pool_ref[:, FOFF:FOFF + OUT_DIM]], axis=1)
    h = lax.dot_general(xm, w3_ref[...], (((1,), (1,)), ((), ())),
                        preferred_element_type=jnp.float32)
    h = h + b3_ref[...]
    m = jnp.mean(h, axis=0, keepdims=True)
    v = jnp.mean((h - m) ** 2, axis=0, keepdims=True)
    h = (h - m) / jnp.sqrt(v + EPS) * g2_ref[...] + bt2_ref[...]
    h = jnp.maximum(h, 0.0)
    o = lax.dot_general(h, w4_ref[...], (((1,), (1,)), ((), ())),
                        preferred_element_type=jnp.float32)
    out_ref[...] = o + b4_ref[...]


def _mlp2(pool, nx, W3, b3, g2, bt2, W4, b4):
    return pl.pallas_call(
        _mlp2_body,
        out_shape=jax.ShapeDtypeStruct((BS_TOT, OUT_DIM), jnp.float32),
    )(pool, nx, W3, b3.reshape(1, OUT_DIM), g2.reshape(1, OUT_DIM),
      bt2.reshape(1, OUT_DIM), W4, b4.reshape(1, OUT_DIM))


def kernel(x_in, W1, b1, g1, bt1, W2, b2, W3, b3, g2, bt2, W4, b4):
    xt = jnp.transpose(x_in, (2, 0, 1))
    tab = _mlp1(xt.reshape(3, BN_TOT), W1, b1, g1, bt1, W2, b2)
    nxt = _fps(xt)
    new_xyz = jnp.transpose(nxt, (1, 0, 2))
    idx = _knn(new_xyz, x_in, jnp.transpose(x_in, (0, 2, 1)))
    pool = _scpool(tab, idx.reshape(NW * NCHUNK, IPC))
    out = _mlp2(pool, new_xyz.reshape(BS_TOT, 3),
                W3, b3, g2, bt2, W4, b4)
    return out.reshape(B, S, OUT_DIM)
